# baseline (speedup 1.0000x reference)
import functools

import jax
import jax.numpy as jnp
from jax import lax
from jax.experimental import pallas as pl
from jax.experimental.pallas import tpu as pltpu, tpu_sc as plsc

N, C, H, W = 4, 256, 128, 128
PH = PW = 7
R = 512
NC, NS = 2, 16
NW = NC * NS
RPW = R // NW
BINS = PH * PW
CB = 8
NCHK = 7
ROWS = CB * 16


def _sc_body(table, rois, out, roiv, yoffA, wyA, xA, wxA, gyT, gxT,
             idxAll, wtAll, rowbuf0, rowbuf1, accR, sem0, sem1):
    wid = lax.axis_index("s") * NC + lax.axis_index("c")
    pltpu.sync_copy(rois.at[pl.ds(wid * (RPW * 5), RPW * 5)], roiv)

    li = lax.iota(jnp.int32, 16)
    sy = (li >> 3) & 1
    cy = (li >> 2) & 1
    sx = (li >> 1) & 1
    cx = li & 1
    ybase = cy * 16 + sy
    xbase = cx * 16 + sx
    fi = li.astype(jnp.float32) * 0.5 + 0.25

    for k in range(4):
        bi = li + 16 * k
        d7 = bi // 7
        gyT[pl.ds(16 * k, 16)] = d7 * 2
        gxT[pl.ds(16 * k, 16)] = (bi - d7 * 7) * 2

    for j in range(BINS * 16 // 16, NCHK * ROWS // 16):
        wtAll[pl.ds(j * 16, 16)] = jnp.zeros((16,), jnp.float32)
        idxAll[j // (ROWS // 16), pl.ds((j % (ROWS // 16)) * 16, 16)] = (
            jnp.zeros((16,), jnp.int32))

    def roi_loop(i, _):
        def param(j):
            return plsc.load_gather(
                roiv, [jnp.full((16,), i * 5 + j, jnp.int32)])

        b = param(0).astype(jnp.int32)
        x1 = param(1) * 0.25 - 0.5
        y1 = param(2) * 0.25 - 0.5
        x2 = param(3) * 0.25 - 0.5
        y2 = param(4) * 0.25 - 0.5
        bHW = b * (H * W)
        zf = jnp.zeros((16,), jnp.float32)

        bin_h = (y2 - y1) / 7.0
        posy = y1 + fi * bin_h
        vy = (posy > -1.0) & (posy < float(H))
        pyc = jnp.clip(posy, 0.0, float(H - 1))
        y0i = pyc.astype(jnp.int32)
        ly = pyc - y0i.astype(jnp.float32)
        hy = 1.0 - ly
        y1i = jnp.minimum(y0i + 1, H - 1)
        hy = jnp.where(vy, hy, zf)
        ly = jnp.where(vy, ly, zf)
        yoffA[pl.ds(0, 16)] = bHW + y0i * W
        yoffA[pl.ds(16, 16)] = bHW + y1i * W
        wyA[pl.ds(0, 16)] = hy
        wyA[pl.ds(16, 16)] = ly

        bin_w = (x2 - x1) / 7.0
        posx = x1 + fi * bin_w
        vx = (posx > -1.0) & (posx < float(W))
        pxc = jnp.clip(posx, 0.0, float(W - 1))
        x0i = pxc.astype(jnp.int32)
        lx = pxc - x0i.astype(jnp.float32)
        hx = 1.0 - lx
        x1i = jnp.minimum(x0i + 1, W - 1)
        hx = jnp.where(vx, hx, zf)
        lx = jnp.where(vx, lx, zf)
        xA[pl.ds(0, 16)] = x0i
        xA[pl.ds(16, 16)] = x1i
        wxA[pl.ds(0, 16)] = hx
        wxA[pl.ds(16, 16)] = lx

        for kc in range(NCHK):
            def prep_bin(j, _, kc=kc):
                bi = kc * CB + j
                gyv = plsc.load_gather(gyT, [jnp.zeros((16,), jnp.int32) + bi])
                gxv = plsc.load_gather(gxT, [jnp.zeros((16,), jnp.int32) + bi])
                ylv = ybase + gyv
                xlv = xbase + gxv
                idx = (plsc.load_gather(yoffA, [ylv])
                       + plsc.load_gather(xA, [xlv]))
                wt = (plsc.load_gather(wyA, [ylv])
                      * plsc.load_gather(wxA, [xlv]) * 0.25)
                idxAll[kc, pl.ds(j * 16, 16)] = idx
                wtAll[pl.ds(kc * ROWS + j * 16, 16)] = wt
                return 0

            lax.fori_loop(0, min(CB, BINS - kc * CB), prep_bin, 0)

        sems = (sem0, sem1)
        bufs = (rowbuf0, rowbuf1)
        cps = [None, None]
        cps[0] = pltpu.async_copy(table.at[idxAll.at[0]], bufs[0], sems[0])
        for k in range(NCHK):
            par = k % 2
            if k + 1 < NCHK:
                cps[(k + 1) % 2] = pltpu.async_copy(
                    table.at[idxAll.at[k + 1]],
                    bufs[(k + 1) % 2], sems[(k + 1) % 2])
            cps[par].wait()
            buf = bufs[par]

            def bin_body(j, _):
                base = k * ROWS + j * 16

                def row_body(lr, acc):
                    wl = plsc.load_gather(
                        wtAll, [jnp.zeros((16,), jnp.int32) + (base + lr)])
                    return tuple(
                        acc[c] + wl * buf[j * 16 + lr, pl.ds(c * 16, 16)]
                        for c in range(16))

                acc = lax.fori_loop(
                    0, 16, row_body,
                    tuple(jnp.zeros((16,), jnp.float32) for _ in range(16)),
                    unroll=4)
                boff = (k * CB + j) * C
                for c in range(16):
                    accR[pl.ds(boff + c * 16, 16)] = acc[c]
                return 0

            nb = min(CB, BINS - k * CB)
            lax.fori_loop(0, nb, bin_body, 0)

        pltpu.sync_copy(accR.at[pl.ds(0, BINS * C)], out.at[wid * RPW + i])
        return 0

    lax.fori_loop(0, RPW, roi_loop, 0)


_sc_call = pl.kernel(
    _sc_body,
    out_type=jax.ShapeDtypeStruct((R, BINS * C), jnp.float32),
    mesh=plsc.VectorSubcoreMesh(core_axis_name="c", subcore_axis_name="s"),
    scratch_types=[
        pltpu.VMEM((RPW * 5,), jnp.float32),
        pltpu.VMEM((32,), jnp.int32),
        pltpu.VMEM((32,), jnp.float32),
        pltpu.VMEM((32,), jnp.int32),
        pltpu.VMEM((32,), jnp.float32),
        pltpu.VMEM((64,), jnp.int32),
        pltpu.VMEM((64,), jnp.int32),
        pltpu.VMEM((NCHK, ROWS), jnp.int32),
        pltpu.VMEM((NCHK * ROWS,), jnp.float32),
        pltpu.VMEM((ROWS, C), jnp.float32),
        pltpu.VMEM((ROWS, C), jnp.float32),
        pltpu.VMEM((BINS * C,), jnp.float32),
        pltpu.SemaphoreType.DMA,
        pltpu.SemaphoreType.DMA,
    ],
    compiler_params=pltpu.CompilerParams(needs_layout_passes=False),
)


@jax.jit
def kernel(input, rois):
    table = jnp.transpose(input, (0, 2, 3, 1)).reshape(N * H * W, C)
    out = _sc_call(table, rois.reshape(-1))
    return out.reshape(R, PH, PW, C).transpose(0, 3, 1, 2)


# speedup vs baseline: 6.7562x; 6.7562x over previous
import functools

import jax
import jax.numpy as jnp
from jax import lax
from jax.experimental import pallas as pl
from jax.experimental.pallas import tpu as pltpu, tpu_sc as plsc

N, C, H, W = 4, 256, 128, 128
PH = PW = 7
R = 512
NC, NS = 2, 16
NW = NC * NS
RPW = R // NW
BINS = PH * PW
CB = 8
NCHK = 7
ROWS = CB * 16


def _sc_body(table, rois, out, roiv, yoffA, wyA, xA, wxA, wtS,
             rowbuf0, rowbuf1, accR, sem0, sem1):
    wid = lax.axis_index("s") * NC + lax.axis_index("c")
    pltpu.sync_copy(rois.at[pl.ds(wid * (RPW * 5), RPW * 5)], roiv)

    li = lax.iota(jnp.int32, 16)
    sy = (li >> 3) & 1
    cy = (li >> 2) & 1
    sx = (li >> 1) & 1
    cx = li & 1
    ybase = cy * 16 + sy
    xbase = cx * 16 + sx
    fi = li.astype(jnp.float32) * 0.5 + 0.25

    def roi_loop(i, _):
        def param(j):
            return plsc.load_gather(
                roiv, [jnp.full((16,), i * 5 + j, jnp.int32)])

        b = param(0).astype(jnp.int32)
        x1 = param(1) * 0.25 - 0.5
        y1 = param(2) * 0.25 - 0.5
        x2 = param(3) * 0.25 - 0.5
        y2 = param(4) * 0.25 - 0.5
        bHW = b * (H * W)
        zf = jnp.zeros((16,), jnp.float32)

        bin_h = (y2 - y1) / 7.0
        posy = y1 + fi * bin_h
        vy = (posy > -1.0) & (posy < float(H))
        pyc = jnp.clip(posy, 0.0, float(H - 1))
        y0i = pyc.astype(jnp.int32)
        ly = pyc - y0i.astype(jnp.float32)
        hy = 1.0 - ly
        y1i = jnp.minimum(y0i + 1, H - 1)
        hy = jnp.where(vy, hy, zf)
        ly = jnp.where(vy, ly, zf)
        yoffA[pl.ds(0, 16)] = bHW + y0i * W
        yoffA[pl.ds(16, 16)] = bHW + y1i * W
        wyA[pl.ds(0, 16)] = hy
        wyA[pl.ds(16, 16)] = ly

        bin_w = (x2 - x1) / 7.0
        posx = x1 + fi * bin_w
        vx = (posx > -1.0) & (posx < float(W))
        pxc = jnp.clip(posx, 0.0, float(W - 1))
        x0i = pxc.astype(jnp.int32)
        lx = pxc - x0i.astype(jnp.float32)
        hx = 1.0 - lx
        x1i = jnp.minimum(x0i + 1, W - 1)
        hx = jnp.where(vx, hx, zf)
        lx = jnp.where(vx, lx, zf)
        xA[pl.ds(0, 16)] = x0i
        xA[pl.ds(16, 16)] = x1i
        wxA[pl.ds(0, 16)] = hx
        wxA[pl.ds(16, 16)] = lx

        sems = (sem0, sem1)
        bufs = (rowbuf0, rowbuf1)

        def fire(g):
            par = g % 2
            cps = []
            for j in range(min(CB, BINS - g * CB)):
                bi = g * CB + j
                ylv = ybase + 2 * (bi // 7)
                xlv = xbase + 2 * (bi % 7)
                idx = (plsc.load_gather(yoffA, [ylv])
                       + plsc.load_gather(xA, [xlv]))
                wt = (plsc.load_gather(wyA, [ylv])
                      * plsc.load_gather(wxA, [xlv]) * 0.25)
                wtS[pl.ds((par * CB + j) * 16, 16)] = wt
                cps.append(pltpu.async_copy(
                    table.at[idx], bufs[par].at[pl.ds(j * 16, 16)],
                    sems[par]))
            return cps

        pend = {0: fire(0)}
        for g in range(NCHK):
            par = g % 2
            if g + 1 < NCHK:
                pend[g + 1] = fire(g + 1)
            for cp in pend.pop(g):
                cp.wait()
            buf = bufs[par]

            def bin_body(j, _, g=g, par=par, buf=buf):
                wbase = par * CB * 16 + j * 16

                def row_body(lr, acc):
                    wl = plsc.load_gather(
                        wtS, [jnp.zeros((16,), jnp.int32) + (wbase + lr)])
                    return tuple(
                        acc[c] + wl * buf[j * 16 + lr, pl.ds(c * 16, 16)]
                        for c in range(16))

                acc = lax.fori_loop(
                    0, 16, row_body,
                    tuple(jnp.zeros((16,), jnp.float32) for _ in range(16)),
                    unroll=4)
                boff = (g * CB + j) * C
                for c in range(16):
                    accR[pl.ds(boff + c * 16, 16)] = acc[c]
                return 0

            lax.fori_loop(0, min(CB, BINS - g * CB), bin_body, 0)

        pltpu.sync_copy(accR.at[pl.ds(0, BINS * C)], out.at[wid * RPW + i])
        return 0

    lax.fori_loop(0, RPW, roi_loop, 0)


_sc_call = pl.kernel(
    _sc_body,
    out_type=jax.ShapeDtypeStruct((R, BINS * C), jnp.float32),
    mesh=plsc.VectorSubcoreMesh(core_axis_name="c", subcore_axis_name="s"),
    scratch_types=[
        pltpu.VMEM((RPW * 5,), jnp.float32),
        pltpu.VMEM((32,), jnp.int32),
        pltpu.VMEM((32,), jnp.float32),
        pltpu.VMEM((32,), jnp.int32),
        pltpu.VMEM((32,), jnp.float32),
        pltpu.VMEM((2 * CB * 16,), jnp.float32),
        pltpu.VMEM((ROWS, C), jnp.float32),
        pltpu.VMEM((ROWS, C), jnp.float32),
        pltpu.VMEM((BINS * C,), jnp.float32),
        pltpu.SemaphoreType.DMA,
        pltpu.SemaphoreType.DMA,
    ],
    compiler_params=pltpu.CompilerParams(needs_layout_passes=False),
)


@jax.jit
def kernel(input, rois):
    table = jnp.transpose(input, (0, 2, 3, 1)).reshape(N * H * W, C)
    out = _sc_call(table, rois.reshape(-1))
    return out.reshape(R, PH, PW, C).transpose(0, 3, 1, 2)
